# baseline (speedup 1.0000x reference)
import functools

import jax
import jax.numpy as jnp
from jax import lax
from jax.experimental import pallas as pl
from jax.experimental.pallas import tpu as pltpu
from jax.experimental.pallas import tpu_sc as plsc

N_NODE = 50000
HID = 128
HEADS = 8
DH = 16
NEG = 0.2

E_EDGE = 600000
NTILE = 16
NCORE = 2
SB = 1024
BLKS = 38
SHARD = SB * BLKS
E_PAD = SHARD * NTILE
PB = 128
C = 6272
CP = 6400
RPT = CP // NTILE
NCHUNK = 8


def _sc_conv_kernel(src_hbm, dst_hbm, h_hbm, as_hbm, ad_hbm,
                    msg_out, den_out,
                    acc_m, acc_d, sv, dv, csrc, crel,
                    idx2a, idx2b, hrowsa, hrowsb, axsa, axsb, axda, axdb,
                    exr, orow,
                    sa1, sa2, sa3, sb1, sb2, sb3):
    c = lax.axis_index("c")
    s = lax.axis_index("s")
    estart = s * SHARD
    r0 = s * RPT
    zvec = jnp.zeros((16,), jnp.float32)
    nd_m1 = ad_hbm.shape[0] - 1
    bsa = (idx2a, hrowsa, axsa, axda, sa1, sa2, sa3)
    bsb = (idx2b, hrowsb, axsb, axdb, sb1, sb2, sb3)

    def zero_all():
        @plsc.parallel_loop(0, PB, 1, unroll=4)
        def zero_rows(i):
            for j in range(HEADS):
                hrowsa[i, pl.ds(j * 16, 16)] = zvec
            exr[i, :] = zvec

    def pass_body(p, _):
        q = 2 * p + c
        base = q * C

        def fire(bs, off):
            idx2, hr, ax1, ax2, s1, s2, s3 = bs
            for j in range(PB // 16):
                r = crel[pl.ds(off + j * 16, 16)]
                idx2[0, pl.ds(j * 16, 16)] = r
                idx2[1, pl.ds(j * 16, 16)] = csrc[pl.ds(off + j * 16, 16)]
                idx2[2, pl.ds(j * 16, 16)] = jnp.minimum(r + base, nd_m1)
            pltpu.async_copy(h_hbm.at[idx2.at[1]], hr, s1)
            pltpu.async_copy(as_hbm.at[idx2.at[1]], ax1, s2)
            pltpu.async_copy(ad_hbm.at[idx2.at[2]], ax2, s3)

        def fire_dummy(bs):
            idx2, hr, ax1, ax2, s1, s2, s3 = bs
            dum = jnp.full((16,), C, jnp.int32)
            zi = jnp.zeros((16,), jnp.int32)
            for j in range(PB // 16):
                idx2[0, pl.ds(j * 16, 16)] = dum
                idx2[1, pl.ds(j * 16, 16)] = zi
                idx2[2, pl.ds(j * 16, 16)] = zi
            pltpu.async_copy(h_hbm.at[idx2.at[1]], hr, s1)
            pltpu.async_copy(as_hbm.at[idx2.at[1]], ax1, s2)
            pltpu.async_copy(ad_hbm.at[idx2.at[2]], ax2, s3)

        def drain(bs):
            idx2, hr, ax1, ax2, s1, s2, s3 = bs
            pltpu.make_async_copy(h_hbm.at[idx2.at[1]], hr, s1).wait()
            pltpu.make_async_copy(as_hbm.at[idx2.at[1]], ax1, s2).wait()
            pltpu.make_async_copy(ad_hbm.at[idx2.at[2]], ax2, s3).wait()

            @plsc.parallel_loop(0, PB, 1, unroll=4)
            def edge_body(e):
                a = ax1[e, :] + ax2[e, :]
                a = jnp.maximum(a, a * NEG)
                ex = jnp.exp(a)
                exr[e, :] = ex
                for h in range(HEADS):
                    orow[e, pl.ds(h * 16, 16)] = (
                        hr[e, pl.ds(h * 16, 16)] * ex[h])

            pltpu.sync_copy(orow, acc_m.at[idx2.at[0]], add=True)
            pltpu.sync_copy(exr, acc_d.at[idx2.at[0]], add=True)

        def fire_p(par, off):
            @pl.when(par == 0)
            def _():
                fire(bsa, off)

            @pl.when(par == 1)
            def _():
                fire(bsb, off)

        def drain_p(sel):
            @pl.when(sel == 0)
            def _():
                drain(bsa)

            @pl.when(sel == 1)
            def _():
                drain(bsb)

        zero_all()
        for t in range(3):
            pltpu.sync_copy(hrowsa, acc_m.at[pl.ds(r0 + t * PB, PB)])
            pltpu.sync_copy(exr, acc_d.at[pl.ds(r0 + t * PB, PB)])
        pltpu.sync_copy(hrowsa.at[pl.ds(0, 16)], acc_m.at[pl.ds(r0 + 3 * PB, 16)])
        pltpu.sync_copy(exr.at[pl.ds(0, 16)], acc_d.at[pl.ds(r0 + 3 * PB, 16)])
        plsc.subcore_barrier()

        fire_dummy(bsa)

        def scan_vec(i, cnt):
            s16 = sv[pl.ds(i * 16, 16)]
            d16 = dv[pl.ds(i * 16, 16)]
            rel = d16 - base
            m = (rel >= 0) & (rel < C)
            plsc.store_compressed(csrc.at[pl.ds(cnt, 16)], s16, mask=m)
            plsc.store_compressed(crel.at[pl.ds(cnt, 16)], rel, mask=m)
            pc = plsc.all_reduce_population_count(m)
            return cnt + pc[0]

        def blk_body(blk, carry):
            cnt, par = carry
            pltpu.sync_copy(src_hbm.at[pl.ds(estart + blk * SB, SB)], sv)
            pltpu.sync_copy(dst_hbm.at[pl.ds(estart + blk * SB, SB)], dv)
            cnt = lax.fori_loop(0, SB // 16, scan_vec, cnt)
            nfull = cnt // PB

            def grp_body(g, par):
                fire_p(par, g * PB)
                drain_p(1 - par)
                return 1 - par

            par = lax.fori_loop(0, nfull, grp_body, par)
            rs = nfull * PB
            for j in range(PB // 16):
                v1 = csrc[pl.ds(rs + j * 16, 16)]
                v2 = crel[pl.ds(rs + j * 16, 16)]
                csrc[pl.ds(j * 16, 16)] = v1
                crel[pl.ds(j * 16, 16)] = v2
            return (cnt - rs, par)

        cnt, par = lax.fori_loop(0, BLKS, blk_body, (0, 1))

        ones_m = jnp.ones((16,), jnp.bool_)
        zero_i = jnp.zeros((16,), jnp.int32)
        dum_r = jnp.full((16,), C, jnp.int32)
        plsc.store_compressed(csrc.at[pl.ds(cnt, 16)], zero_i, mask=ones_m)
        plsc.store_compressed(crel.at[pl.ds(cnt, 16)], dum_r, mask=ones_m)
        for j in range(PB // 16):

            @pl.when(j * 16 >= cnt)
            def _():
                csrc[pl.ds(j * 16, 16)] = zero_i
                crel[pl.ds(j * 16, 16)] = dum_r

        fire_p(par, 0)
        drain_p(1 - par)
        drain_p(par)
        plsc.subcore_barrier()

        for t in range(3):
            pltpu.sync_copy(acc_m.at[pl.ds(r0 + t * PB, PB)],
                            msg_out.at[q, pl.ds(r0 + t * PB, PB)])
            pltpu.sync_copy(acc_d.at[pl.ds(r0 + t * PB, PB)],
                            den_out.at[q, pl.ds(r0 + t * PB, PB)])
        pltpu.sync_copy(acc_m.at[pl.ds(r0 + 3 * PB, 16)],
                        msg_out.at[q, pl.ds(r0 + 3 * PB, 16)])
        pltpu.sync_copy(acc_d.at[pl.ds(r0 + 3 * PB, 16)],
                        den_out.at[q, pl.ds(r0 + 3 * PB, 16)])
        plsc.subcore_barrier()
        return 0

    lax.fori_loop(0, NCHUNK // 2, pass_body, 0)


@functools.partial(
    pl.kernel,
    out_type=(jax.ShapeDtypeStruct((NCHUNK, CP, HID), jnp.float32),
              jax.ShapeDtypeStruct((NCHUNK, CP, 16), jnp.float32)),
    mesh=plsc.VectorSubcoreMesh(core_axis_name="c", subcore_axis_name="s"),
    compiler_params=pltpu.CompilerParams(
        needs_layout_passes=False, use_tc_tiling_on_sc=False),
    scratch_types=(
        pltpu.VMEM_SHARED((CP, HID), jnp.float32),
        pltpu.VMEM_SHARED((CP, 16), jnp.float32),
        pltpu.VMEM((SB,), jnp.int32),
        pltpu.VMEM((SB,), jnp.int32),
        pltpu.VMEM((SB + 2 * PB,), jnp.int32),
        pltpu.VMEM((SB + 2 * PB,), jnp.int32),
        pltpu.VMEM((8, PB), jnp.int32),
        pltpu.VMEM((8, PB), jnp.int32),
        pltpu.VMEM((PB, HID), jnp.float32),
        pltpu.VMEM((PB, HID), jnp.float32),
        pltpu.VMEM((PB, 16), jnp.float32),
        pltpu.VMEM((PB, 16), jnp.float32),
        pltpu.VMEM((PB, 16), jnp.float32),
        pltpu.VMEM((PB, 16), jnp.float32),
        pltpu.VMEM((PB, 16), jnp.float32),
        pltpu.VMEM((PB, HID), jnp.float32),
        pltpu.SemaphoreType.DMA,
        pltpu.SemaphoreType.DMA,
        pltpu.SemaphoreType.DMA,
        pltpu.SemaphoreType.DMA,
        pltpu.SemaphoreType.DMA,
        pltpu.SemaphoreType.DMA,
    ),
)
def _sc_conv(src_hbm, dst_hbm, h_hbm, as_hbm, ad_hbm, msg_out, den_out, *rest):
    _sc_conv_kernel(src_hbm, dst_hbm, h_hbm, as_hbm, ad_hbm,
                    msg_out, den_out, *rest)


def _proj_body(x_ref, w_ref, b_ref, a1_ref, a2_ref, h_ref, o1_ref, o2_ref):
    h = jnp.dot(x_ref[...], w_ref[...], preferred_element_type=jnp.float32)
    h = h + b_ref[...]
    h_ref[...] = h
    o1_ref[...] = jnp.dot(h, a1_ref[...], preferred_element_type=jnp.float32)
    o2_ref[...] = jnp.dot(h, a2_ref[...], preferred_element_type=jnp.float32)


def _proj(x, w, b, a1, a2):
    rb = 1000
    grid = (x.shape[0] // rb,)
    return pl.pallas_call(
        _proj_body,
        grid=grid,
        in_specs=[
            pl.BlockSpec((rb, HID), lambda i: (i, 0)),
            pl.BlockSpec((HID, HID), lambda i: (0, 0)),
            pl.BlockSpec((1, HID), lambda i: (0, 0)),
            pl.BlockSpec((HID, 16), lambda i: (0, 0)),
            pl.BlockSpec((HID, 16), lambda i: (0, 0)),
        ],
        out_specs=[
            pl.BlockSpec((rb, HID), lambda i: (i, 0)),
            pl.BlockSpec((rb, 16), lambda i: (i, 0)),
            pl.BlockSpec((rb, 16), lambda i: (i, 0)),
        ],
        out_shape=[
            jax.ShapeDtypeStruct((x.shape[0], HID), jnp.float32),
            jax.ShapeDtypeStruct((x.shape[0], 16), jnp.float32),
            jax.ShapeDtypeStruct((x.shape[0], 16), jnp.float32),
        ],
    )(x, w, b, a1, a2)


def _fin_body(m_ref, d_ref, r_ref, o_ref):
    den16 = jnp.dot(d_ref[0], r_ref[...], preferred_element_type=jnp.float32)
    o_ref[0] = jnp.maximum(m_ref[0] / (den16 + 1e-16), 0.0)


def _finalize(msg3, den3, rmat):
    rb = 640
    grid = (NCHUNK, CP // rb)
    out3 = pl.pallas_call(
        _fin_body,
        grid=grid,
        in_specs=[
            pl.BlockSpec((1, rb, HID), lambda i, j: (i, j, 0)),
            pl.BlockSpec((1, rb, 16), lambda i, j: (i, j, 0)),
            pl.BlockSpec((16, HID), lambda i, j: (0, 0)),
        ],
        out_specs=pl.BlockSpec((1, rb, HID), lambda i, j: (i, j, 0)),
        out_shape=jax.ShapeDtypeStruct((NCHUNK, CP, HID), jnp.float32),
    )(msg3, den3, rmat)
    return out3[:, :C, :].reshape(NCHUNK * C, HID)[:N_NODE]


def _att_mat(att):
    blk = jnp.eye(HEADS, dtype=att.dtype)[:, None, :] * att[:, :, None]
    return jnp.pad(blk.reshape(HID, HEADS), ((0, 0), (0, 16 - HEADS)))


def _pad_edges(ei):
    src = jnp.concatenate(
        [ei[0], jnp.zeros((E_PAD - E_EDGE,), jnp.int32)])
    dst = jnp.concatenate(
        [ei[1], jnp.full((E_PAD - E_EDGE,), jnp.int32(2 ** 30))])
    return src, dst


def kernel(x_author, x_paper, edge_index_writes, edge_index_rev,
           W_proj_author, b_proj_author, W_proj_paper, b_proj_paper,
           att_src_writes, att_dst_writes, att_src_rev, att_dst_rev,
           q_sem, W_k_sem, b_k_sem):
    h_a, aa_w, aa_r = _proj(x_author, W_proj_author, b_proj_author.reshape(1, HID),
                            _att_mat(att_src_writes), _att_mat(att_dst_rev))
    h_p, ap_w, ap_r = _proj(x_paper, W_proj_paper, b_proj_paper.reshape(1, HID),
                            _att_mat(att_dst_writes), _att_mat(att_src_rev))

    src_w, dst_w = _pad_edges(edge_index_writes)
    src_r, dst_r = _pad_edges(edge_index_rev)

    msg_w, den_w = _sc_conv(src_w, dst_w, h_a, aa_w, ap_w)
    msg_r, den_r = _sc_conv(src_r, dst_r, h_p, ap_r, aa_r)

    rmat = jnp.pad(jnp.repeat(jnp.eye(HEADS, dtype=jnp.float32), 16, axis=1),
                   ((0, 16 - HEADS), (0, 0)))
    out_paper = _finalize(msg_w, den_w, rmat)
    out_author = _finalize(msg_r, den_r, rmat)
    return (out_author, out_paper)


# speedup vs baseline: 1.5810x; 1.5810x over previous
import functools

import jax
import jax.numpy as jnp
from jax import lax
from jax.experimental import pallas as pl
from jax.experimental.pallas import tpu as pltpu
from jax.experimental.pallas import tpu_sc as plsc

N_NODE = 50000
HID = 128
HEADS = 8
DH = 16
NEG = 0.2

E_EDGE = 600000
NTILE = 16
NCORE = 2
SB = 1024
BLKS = 38
SHARD = SB * BLKS
E_PAD = SHARD * NTILE
PB = 128
C = 8400
CP = 8448
RPT = CP // NTILE
NCHUNK = 6


def _sc_conv_kernel(src_hbm, dst_hbm, h_hbm, as_hbm, ad_hbm,
                    msg_out, den_out,
                    acc_m, acc_d, sv, dv, csrc, crel, idx2,
                    hrows, axs, axd, exr, orow, sem1, sem2, sem3):
    c = lax.axis_index("c")
    s = lax.axis_index("s")
    estart = s * SHARD
    r0 = s * RPT
    zvec = jnp.zeros((16,), jnp.float32)
    nd_m1 = ad_hbm.shape[0] - 1

    def zero_all():
        @plsc.parallel_loop(0, PB, 1, unroll=4)
        def zero_rows(i):
            for j in range(HEADS):
                orow[i, pl.ds(j * 16, 16)] = zvec
            exr[i, :] = zvec

    for p in range(NCHUNK // 2):
        q = 2 * p + c
        base = q * C

        def group(off):
            for j in range(PB // 16):
                r = crel[pl.ds(off + j * 16, 16)]
                idx2[0, pl.ds(j * 16, 16)] = r
                idx2[1, pl.ds(j * 16, 16)] = csrc[pl.ds(off + j * 16, 16)]
                idx2[2, pl.ds(j * 16, 16)] = jnp.minimum(r + base, nd_m1)
            c1 = pltpu.async_copy(h_hbm.at[idx2.at[1]], hrows, sem1)
            c2 = pltpu.async_copy(as_hbm.at[idx2.at[1]], axs, sem2)
            c3 = pltpu.async_copy(ad_hbm.at[idx2.at[2]], axd, sem3)
            c1.wait()
            c2.wait()
            c3.wait()

            @plsc.parallel_loop(0, PB, 1, unroll=4)
            def edge_body(e):
                a = axs[e, :] + axd[e, :]
                a = jnp.maximum(a, a * NEG)
                ex = jnp.exp(a)
                exr[e, :] = ex
                for h in range(HEADS):
                    orow[e, pl.ds(h * 16, 16)] = (
                        hrows[e, pl.ds(h * 16, 16)] * ex[h])

            pltpu.sync_copy(orow, acc_m.at[idx2.at[0]], add=True)
            pltpu.sync_copy(exr, acc_d.at[idx2.at[0]], add=True)

        zero_all()
        for t in range(4):
            pltpu.sync_copy(orow, acc_m.at[pl.ds(r0 + t * PB, PB)])
            pltpu.sync_copy(exr, acc_d.at[pl.ds(r0 + t * PB, PB)])
        pltpu.sync_copy(orow.at[pl.ds(0, 16)], acc_m.at[pl.ds(r0 + 4 * PB, 16)])
        pltpu.sync_copy(exr.at[pl.ds(0, 16)], acc_d.at[pl.ds(r0 + 4 * PB, 16)])
        plsc.subcore_barrier()

        def scan_vec(i, cnt):
            s16 = sv[pl.ds(i * 16, 16)]
            d16 = dv[pl.ds(i * 16, 16)]
            rel = d16 - base
            m = (rel >= 0) & (rel < C)
            plsc.store_compressed(csrc.at[pl.ds(cnt, 16)], s16, mask=m)
            plsc.store_compressed(crel.at[pl.ds(cnt, 16)], rel, mask=m)
            pc = plsc.all_reduce_population_count(m)
            return cnt + pc[0]

        def blk_body(blk, cnt):
            pltpu.sync_copy(src_hbm.at[pl.ds(estart + blk * SB, SB)], sv)
            pltpu.sync_copy(dst_hbm.at[pl.ds(estart + blk * SB, SB)], dv)
            cnt = lax.fori_loop(0, SB // 16, scan_vec, cnt)
            nfull = cnt // PB

            def grp_body(g, _):
                group(g * PB)
                return 0

            lax.fori_loop(0, nfull, grp_body, 0)
            rs = nfull * PB
            for j in range(PB // 16):
                v1 = csrc[pl.ds(rs + j * 16, 16)]
                v2 = crel[pl.ds(rs + j * 16, 16)]
                csrc[pl.ds(j * 16, 16)] = v1
                crel[pl.ds(j * 16, 16)] = v2
            return cnt - rs

        cnt = lax.fori_loop(0, BLKS, blk_body, 0)

        ones_m = jnp.ones((16,), jnp.bool_)
        zero_i = jnp.zeros((16,), jnp.int32)
        dum_r = jnp.full((16,), C, jnp.int32)
        plsc.store_compressed(csrc.at[pl.ds(cnt, 16)], zero_i, mask=ones_m)
        plsc.store_compressed(crel.at[pl.ds(cnt, 16)], dum_r, mask=ones_m)
        for j in range(PB // 16):

            @pl.when(j * 16 >= cnt)
            def _():
                csrc[pl.ds(j * 16, 16)] = zero_i
                crel[pl.ds(j * 16, 16)] = dum_r

        group(0)
        plsc.subcore_barrier()

        for t in range(4):
            pltpu.sync_copy(acc_m.at[pl.ds(r0 + t * PB, PB)],
                            msg_out.at[q, pl.ds(r0 + t * PB, PB)])
            pltpu.sync_copy(acc_d.at[pl.ds(r0 + t * PB, PB)],
                            den_out.at[q, pl.ds(r0 + t * PB, PB)])
        pltpu.sync_copy(acc_m.at[pl.ds(r0 + 4 * PB, 16)],
                        msg_out.at[q, pl.ds(r0 + 4 * PB, 16)])
        pltpu.sync_copy(acc_d.at[pl.ds(r0 + 4 * PB, 16)],
                        den_out.at[q, pl.ds(r0 + 4 * PB, 16)])
        plsc.subcore_barrier()


@functools.partial(
    pl.kernel,
    out_type=(jax.ShapeDtypeStruct((NCHUNK, CP, HID), jnp.float32),
              jax.ShapeDtypeStruct((NCHUNK, CP, 16), jnp.float32)),
    mesh=plsc.VectorSubcoreMesh(core_axis_name="c", subcore_axis_name="s"),
    compiler_params=pltpu.CompilerParams(
        needs_layout_passes=False, use_tc_tiling_on_sc=False),
    scratch_types=(
        pltpu.VMEM_SHARED((CP, HID), jnp.float32),
        pltpu.VMEM_SHARED((CP, 16), jnp.float32),
        pltpu.VMEM((SB,), jnp.int32),
        pltpu.VMEM((SB,), jnp.int32),
        pltpu.VMEM((SB + 2 * PB,), jnp.int32),
        pltpu.VMEM((SB + 2 * PB,), jnp.int32),
        pltpu.VMEM((8, PB), jnp.int32),
        pltpu.VMEM((PB, HID), jnp.float32),
        pltpu.VMEM((PB, 16), jnp.float32),
        pltpu.VMEM((PB, 16), jnp.float32),
        pltpu.VMEM((PB, 16), jnp.float32),
        pltpu.VMEM((PB, HID), jnp.float32),
        pltpu.SemaphoreType.DMA,
        pltpu.SemaphoreType.DMA,
        pltpu.SemaphoreType.DMA,
    ),
)
def _sc_conv(src_hbm, dst_hbm, h_hbm, as_hbm, ad_hbm, msg_out, den_out, *rest):
    _sc_conv_kernel(src_hbm, dst_hbm, h_hbm, as_hbm, ad_hbm,
                    msg_out, den_out, *rest)


def _proj_body(x_ref, w_ref, b_ref, a1_ref, a2_ref, h_ref, o1_ref, o2_ref):
    h = jnp.dot(x_ref[...], w_ref[...], preferred_element_type=jnp.float32)
    h = h + b_ref[...]
    h_ref[...] = h
    o1_ref[...] = jnp.dot(h, a1_ref[...], preferred_element_type=jnp.float32)
    o2_ref[...] = jnp.dot(h, a2_ref[...], preferred_element_type=jnp.float32)


def _proj(x, w, b, a1, a2):
    rb = 1000
    grid = (x.shape[0] // rb,)
    return pl.pallas_call(
        _proj_body,
        grid=grid,
        in_specs=[
            pl.BlockSpec((rb, HID), lambda i: (i, 0)),
            pl.BlockSpec((HID, HID), lambda i: (0, 0)),
            pl.BlockSpec((1, HID), lambda i: (0, 0)),
            pl.BlockSpec((HID, 16), lambda i: (0, 0)),
            pl.BlockSpec((HID, 16), lambda i: (0, 0)),
        ],
        out_specs=[
            pl.BlockSpec((rb, HID), lambda i: (i, 0)),
            pl.BlockSpec((rb, 16), lambda i: (i, 0)),
            pl.BlockSpec((rb, 16), lambda i: (i, 0)),
        ],
        out_shape=[
            jax.ShapeDtypeStruct((x.shape[0], HID), jnp.float32),
            jax.ShapeDtypeStruct((x.shape[0], 16), jnp.float32),
            jax.ShapeDtypeStruct((x.shape[0], 16), jnp.float32),
        ],
    )(x, w, b, a1, a2)


def _fin_body(m_ref, d_ref, r_ref, o_ref):
    den16 = jnp.dot(d_ref[0], r_ref[...], preferred_element_type=jnp.float32)
    o_ref[0] = jnp.maximum(m_ref[0] / (den16 + 1e-16), 0.0)


def _finalize(msg3, den3, rmat):
    rb = 704
    grid = (NCHUNK, CP // rb)
    out3 = pl.pallas_call(
        _fin_body,
        grid=grid,
        in_specs=[
            pl.BlockSpec((1, rb, HID), lambda i, j: (i, j, 0)),
            pl.BlockSpec((1, rb, 16), lambda i, j: (i, j, 0)),
            pl.BlockSpec((16, HID), lambda i, j: (0, 0)),
        ],
        out_specs=pl.BlockSpec((1, rb, HID), lambda i, j: (i, j, 0)),
        out_shape=jax.ShapeDtypeStruct((NCHUNK, CP, HID), jnp.float32),
    )(msg3, den3, rmat)
    return out3[:, :C, :].reshape(NCHUNK * C, HID)[:N_NODE]


def _att_mat(att):
    blk = jnp.eye(HEADS, dtype=att.dtype)[:, None, :] * att[:, :, None]
    return jnp.pad(blk.reshape(HID, HEADS), ((0, 0), (0, 16 - HEADS)))


def _pad_edges(ei):
    src = jnp.concatenate(
        [ei[0], jnp.zeros((E_PAD - E_EDGE,), jnp.int32)])
    dst = jnp.concatenate(
        [ei[1], jnp.full((E_PAD - E_EDGE,), jnp.int32(2 ** 30))])
    return src, dst


def kernel(x_author, x_paper, edge_index_writes, edge_index_rev,
           W_proj_author, b_proj_author, W_proj_paper, b_proj_paper,
           att_src_writes, att_dst_writes, att_src_rev, att_dst_rev,
           q_sem, W_k_sem, b_k_sem):
    h_a, aa_w, aa_r = _proj(x_author, W_proj_author, b_proj_author.reshape(1, HID),
                            _att_mat(att_src_writes), _att_mat(att_dst_rev))
    h_p, ap_w, ap_r = _proj(x_paper, W_proj_paper, b_proj_paper.reshape(1, HID),
                            _att_mat(att_dst_writes), _att_mat(att_src_rev))

    src_w, dst_w = _pad_edges(edge_index_writes)
    src_r, dst_r = _pad_edges(edge_index_rev)

    msg_w, den_w = _sc_conv(src_w, dst_w, h_a, aa_w, ap_w)
    msg_r, den_r = _sc_conv(src_r, dst_r, h_p, ap_r, aa_r)

    rmat = jnp.pad(jnp.repeat(jnp.eye(HEADS, dtype=jnp.float32), 16, axis=1),
                   ((0, 16 - HEADS), (0, 0)))
    out_paper = _finalize(msg_w, den_w, rmat)
    out_author = _finalize(msg_r, den_r, rmat)
    return (out_author, out_paper)


# speedup vs baseline: 1.6688x; 1.0555x over previous
import functools

import jax
import jax.numpy as jnp
from jax import lax
from jax.experimental import pallas as pl
from jax.experimental.pallas import tpu as pltpu
from jax.experimental.pallas import tpu_sc as plsc

N_NODE = 50000
HID = 128
HEADS = 8
DH = 16
NEG = 0.2

E_EDGE = 600000
NTILE = 16
NCORE = 2
SB = 2048
BLKS = 19
SHARD = SB * BLKS
E_PAD = SHARD * NTILE
PB = 128
C = 8400
CP = 8448
RPT = CP // NTILE
NCHUNK = 6


def _sc_conv_kernel(src_hbm, dst_hbm, h_hbm, as_hbm, ad_hbm,
                    msg_out, den_out,
                    acc_m, acc_d, sv, dv, csrc, crel, idx2,
                    hrows, axs, axd, exr, orow, sem1, sem2, sem3):
    c = lax.axis_index("c")
    s = lax.axis_index("s")
    estart = s * SHARD
    r0 = s * RPT
    zvec = jnp.zeros((16,), jnp.float32)
    nd_m1 = ad_hbm.shape[0] - 1

    def zero_all():
        @plsc.parallel_loop(0, PB, 1, unroll=4)
        def zero_rows(i):
            for j in range(HEADS):
                orow[i, pl.ds(j * 16, 16)] = zvec
            exr[i, :] = zvec

    for p in range(NCHUNK // 2):
        q = 2 * p + c
        base = q * C

        def group(off):
            for j in range(PB // 16):
                r = crel[pl.ds(off + j * 16, 16)]
                idx2[0, pl.ds(j * 16, 16)] = r
                idx2[1, pl.ds(j * 16, 16)] = csrc[pl.ds(off + j * 16, 16)]
                idx2[2, pl.ds(j * 16, 16)] = jnp.minimum(r + base, nd_m1)
            c1 = pltpu.async_copy(h_hbm.at[idx2.at[1]], hrows, sem1)
            c2 = pltpu.async_copy(as_hbm.at[idx2.at[1]], axs, sem2)
            c3 = pltpu.async_copy(ad_hbm.at[idx2.at[2]], axd, sem3)
            c1.wait()
            c2.wait()
            c3.wait()

            @plsc.parallel_loop(0, PB, 1, unroll=8)
            def edge_body(e):
                a = axs[e, :] + axd[e, :]
                a = jnp.maximum(a, a * NEG)
                ex = jnp.exp(a)
                exr[e, :] = ex
                for h in range(HEADS):
                    orow[e, pl.ds(h * 16, 16)] = (
                        hrows[e, pl.ds(h * 16, 16)] * ex[h])

            pltpu.sync_copy(orow, acc_m.at[idx2.at[0]], add=True)
            pltpu.sync_copy(exr, acc_d.at[idx2.at[0]], add=True)

        zero_all()
        for t in range(4):
            pltpu.sync_copy(orow, acc_m.at[pl.ds(r0 + t * PB, PB)])
            pltpu.sync_copy(exr, acc_d.at[pl.ds(r0 + t * PB, PB)])
        pltpu.sync_copy(orow.at[pl.ds(0, 16)], acc_m.at[pl.ds(r0 + 4 * PB, 16)])
        pltpu.sync_copy(exr.at[pl.ds(0, 16)], acc_d.at[pl.ds(r0 + 4 * PB, 16)])
        plsc.subcore_barrier()

        def scan_vec(i, cnt):
            s16 = sv[pl.ds(i * 16, 16)]
            d16 = dv[pl.ds(i * 16, 16)]
            rel = d16 - base
            m = (rel >= 0) & (rel < C)
            plsc.store_compressed(csrc.at[pl.ds(cnt, 16)], s16, mask=m)
            plsc.store_compressed(crel.at[pl.ds(cnt, 16)], rel, mask=m)
            pc = plsc.all_reduce_population_count(m)
            return cnt + pc[0]

        def blk_body(blk, cnt):
            pltpu.sync_copy(src_hbm.at[pl.ds(estart + blk * SB, SB)], sv)
            pltpu.sync_copy(dst_hbm.at[pl.ds(estart + blk * SB, SB)], dv)
            cnt = lax.fori_loop(0, SB // 16, scan_vec, cnt)
            nfull = cnt // PB

            def grp_body(g, _):
                group(g * PB)
                return 0

            lax.fori_loop(0, nfull, grp_body, 0)
            rs = nfull * PB
            for j in range(PB // 16):
                v1 = csrc[pl.ds(rs + j * 16, 16)]
                v2 = crel[pl.ds(rs + j * 16, 16)]
                csrc[pl.ds(j * 16, 16)] = v1
                crel[pl.ds(j * 16, 16)] = v2
            return cnt - rs

        cnt = lax.fori_loop(0, BLKS, blk_body, 0)

        ones_m = jnp.ones((16,), jnp.bool_)
        zero_i = jnp.zeros((16,), jnp.int32)
        dum_r = jnp.full((16,), C, jnp.int32)
        plsc.store_compressed(csrc.at[pl.ds(cnt, 16)], zero_i, mask=ones_m)
        plsc.store_compressed(crel.at[pl.ds(cnt, 16)], dum_r, mask=ones_m)
        for j in range(PB // 16):

            @pl.when(j * 16 >= cnt)
            def _():
                csrc[pl.ds(j * 16, 16)] = zero_i
                crel[pl.ds(j * 16, 16)] = dum_r

        group(0)
        plsc.subcore_barrier()

        for t in range(4):
            pltpu.sync_copy(acc_m.at[pl.ds(r0 + t * PB, PB)],
                            msg_out.at[q, pl.ds(r0 + t * PB, PB)])
            pltpu.sync_copy(acc_d.at[pl.ds(r0 + t * PB, PB)],
                            den_out.at[q, pl.ds(r0 + t * PB, PB)])
        pltpu.sync_copy(acc_m.at[pl.ds(r0 + 4 * PB, 16)],
                        msg_out.at[q, pl.ds(r0 + 4 * PB, 16)])
        pltpu.sync_copy(acc_d.at[pl.ds(r0 + 4 * PB, 16)],
                        den_out.at[q, pl.ds(r0 + 4 * PB, 16)])
        plsc.subcore_barrier()


@functools.partial(
    pl.kernel,
    out_type=(jax.ShapeDtypeStruct((NCHUNK, CP, HID), jnp.float32),
              jax.ShapeDtypeStruct((NCHUNK, CP, 16), jnp.float32)),
    mesh=plsc.VectorSubcoreMesh(core_axis_name="c", subcore_axis_name="s"),
    compiler_params=pltpu.CompilerParams(
        needs_layout_passes=False, use_tc_tiling_on_sc=False),
    scratch_types=(
        pltpu.VMEM_SHARED((CP, HID), jnp.float32),
        pltpu.VMEM_SHARED((CP, 16), jnp.float32),
        pltpu.VMEM((SB,), jnp.int32),
        pltpu.VMEM((SB,), jnp.int32),
        pltpu.VMEM((SB + 2 * PB,), jnp.int32),
        pltpu.VMEM((SB + 2 * PB,), jnp.int32),
        pltpu.VMEM((8, PB), jnp.int32),
        pltpu.VMEM((PB, HID), jnp.float32),
        pltpu.VMEM((PB, 16), jnp.float32),
        pltpu.VMEM((PB, 16), jnp.float32),
        pltpu.VMEM((PB, 16), jnp.float32),
        pltpu.VMEM((PB, HID), jnp.float32),
        pltpu.SemaphoreType.DMA,
        pltpu.SemaphoreType.DMA,
        pltpu.SemaphoreType.DMA,
    ),
)
def _sc_conv(src_hbm, dst_hbm, h_hbm, as_hbm, ad_hbm, msg_out, den_out, *rest):
    _sc_conv_kernel(src_hbm, dst_hbm, h_hbm, as_hbm, ad_hbm,
                    msg_out, den_out, *rest)


def _proj_body(x_ref, w_ref, b_ref, a1_ref, a2_ref, h_ref, o1_ref, o2_ref):
    h = jnp.dot(x_ref[...], w_ref[...], preferred_element_type=jnp.float32)
    h = h + b_ref[...]
    h_ref[...] = h
    o1_ref[...] = jnp.dot(h, a1_ref[...], preferred_element_type=jnp.float32)
    o2_ref[...] = jnp.dot(h, a2_ref[...], preferred_element_type=jnp.float32)


def _proj(x, w, b, a1, a2):
    rb = 1000
    grid = (x.shape[0] // rb,)
    return pl.pallas_call(
        _proj_body,
        grid=grid,
        in_specs=[
            pl.BlockSpec((rb, HID), lambda i: (i, 0)),
            pl.BlockSpec((HID, HID), lambda i: (0, 0)),
            pl.BlockSpec((1, HID), lambda i: (0, 0)),
            pl.BlockSpec((HID, 16), lambda i: (0, 0)),
            pl.BlockSpec((HID, 16), lambda i: (0, 0)),
        ],
        out_specs=[
            pl.BlockSpec((rb, HID), lambda i: (i, 0)),
            pl.BlockSpec((rb, 16), lambda i: (i, 0)),
            pl.BlockSpec((rb, 16), lambda i: (i, 0)),
        ],
        out_shape=[
            jax.ShapeDtypeStruct((x.shape[0], HID), jnp.float32),
            jax.ShapeDtypeStruct((x.shape[0], 16), jnp.float32),
            jax.ShapeDtypeStruct((x.shape[0], 16), jnp.float32),
        ],
    )(x, w, b, a1, a2)


def _fin_body(m_ref, d_ref, r_ref, o_ref):
    den16 = jnp.dot(d_ref[0], r_ref[...], preferred_element_type=jnp.float32)
    o_ref[0] = jnp.maximum(m_ref[0] / (den16 + 1e-16), 0.0)


def _finalize(msg3, den3, rmat):
    rb = 704
    grid = (NCHUNK, CP // rb)
    out3 = pl.pallas_call(
        _fin_body,
        grid=grid,
        in_specs=[
            pl.BlockSpec((1, rb, HID), lambda i, j: (i, j, 0)),
            pl.BlockSpec((1, rb, 16), lambda i, j: (i, j, 0)),
            pl.BlockSpec((16, HID), lambda i, j: (0, 0)),
        ],
        out_specs=pl.BlockSpec((1, rb, HID), lambda i, j: (i, j, 0)),
        out_shape=jax.ShapeDtypeStruct((NCHUNK, CP, HID), jnp.float32),
    )(msg3, den3, rmat)
    return out3[:, :C, :].reshape(NCHUNK * C, HID)[:N_NODE]


def _att_mat(att):
    blk = jnp.eye(HEADS, dtype=att.dtype)[:, None, :] * att[:, :, None]
    return jnp.pad(blk.reshape(HID, HEADS), ((0, 0), (0, 16 - HEADS)))


def _pad_edges(ei):
    src = jnp.concatenate(
        [ei[0], jnp.zeros((E_PAD - E_EDGE,), jnp.int32)])
    dst = jnp.concatenate(
        [ei[1], jnp.full((E_PAD - E_EDGE,), jnp.int32(2 ** 30))])
    return src, dst


def kernel(x_author, x_paper, edge_index_writes, edge_index_rev,
           W_proj_author, b_proj_author, W_proj_paper, b_proj_paper,
           att_src_writes, att_dst_writes, att_src_rev, att_dst_rev,
           q_sem, W_k_sem, b_k_sem):
    h_a, aa_w, aa_r = _proj(x_author, W_proj_author, b_proj_author.reshape(1, HID),
                            _att_mat(att_src_writes), _att_mat(att_dst_rev))
    h_p, ap_w, ap_r = _proj(x_paper, W_proj_paper, b_proj_paper.reshape(1, HID),
                            _att_mat(att_dst_writes), _att_mat(att_src_rev))

    src_w, dst_w = _pad_edges(edge_index_writes)
    src_r, dst_r = _pad_edges(edge_index_rev)

    msg_w, den_w = _sc_conv(src_w, dst_w, h_a, aa_w, ap_w)
    msg_r, den_r = _sc_conv(src_r, dst_r, h_p, ap_r, aa_r)

    rmat = jnp.pad(jnp.repeat(jnp.eye(HEADS, dtype=jnp.float32), 16, axis=1),
                   ((0, 16 - HEADS), (0, 0)))
    out_paper = _finalize(msg_w, den_w, rmat)
    out_author = _finalize(msg_r, den_r, rmat)
    return (out_author, out_paper)


# speedup vs baseline: 1.6703x; 1.0009x over previous
import functools

import jax
import jax.numpy as jnp
from jax import lax
from jax.experimental import pallas as pl
from jax.experimental.pallas import tpu as pltpu
from jax.experimental.pallas import tpu_sc as plsc

N_NODE = 50000
HID = 128
HEADS = 8
DH = 16
NEG = 0.2

E_EDGE = 600000
NTILE = 16
NCORE = 2
SB = 2048
BLKS = 19
SHARD = SB * BLKS
E_PAD = SHARD * NTILE
PB = 128
C = 8400
CP = 8448
FW = 144
RPT = CP // NTILE
NCHUNK = 6


def _sc_conv_kernel(src_hbm, dst_hbm, h_hbm, as_hbm, ad_hbm,
                    acc_out,
                    acc_m, sv, dv, csrc, crel, idx2,
                    hrows, axs, axd, orow, sem1, sem2, sem3):
    c = lax.axis_index("c")
    s = lax.axis_index("s")
    estart = s * SHARD
    r0 = s * RPT
    zvec = jnp.zeros((16,), jnp.float32)
    nd_m1 = ad_hbm.shape[0] - 1

    def zero_all():
        @plsc.parallel_loop(0, PB, 1, unroll=4)
        def zero_rows(i):
            for j in range(FW // 16):
                orow[i, pl.ds(j * 16, 16)] = zvec

    for p in range(NCHUNK // 2):
        q = 2 * p + c
        base = q * C

        def group(off):
            for j in range(PB // 16):
                r = crel[pl.ds(off + j * 16, 16)]
                idx2[0, pl.ds(j * 16, 16)] = r
                idx2[1, pl.ds(j * 16, 16)] = csrc[pl.ds(off + j * 16, 16)]
                idx2[2, pl.ds(j * 16, 16)] = jnp.minimum(r + base, nd_m1)
            c1 = pltpu.async_copy(h_hbm.at[idx2.at[1]], hrows, sem1)
            c2 = pltpu.async_copy(as_hbm.at[idx2.at[1]], axs, sem2)
            c3 = pltpu.async_copy(ad_hbm.at[idx2.at[2]], axd, sem3)
            c1.wait()
            c2.wait()
            c3.wait()

            @plsc.parallel_loop(0, PB, 1, unroll=8)
            def edge_body(e):
                a = axs[e, :] + axd[e, :]
                a = jnp.maximum(a, a * NEG)
                ex = jnp.exp(a)
                orow[e, pl.ds(HID, 16)] = ex
                for h in range(HEADS):
                    orow[e, pl.ds(h * 16, 16)] = (
                        hrows[e, pl.ds(h * 16, 16)] * ex[h])

            pltpu.sync_copy(orow, acc_m.at[idx2.at[0]], add=True)

        zero_all()
        for t in range(4):
            pltpu.sync_copy(orow, acc_m.at[pl.ds(r0 + t * PB, PB)])
        pltpu.sync_copy(orow.at[pl.ds(0, 16)], acc_m.at[pl.ds(r0 + 4 * PB, 16)])
        plsc.subcore_barrier()

        def scan_vec(i, cnt):
            s16 = sv[pl.ds(i * 16, 16)]
            d16 = dv[pl.ds(i * 16, 16)]
            rel = d16 - base
            m = (rel >= 0) & (rel < C)
            plsc.store_compressed(csrc.at[pl.ds(cnt, 16)], s16, mask=m)
            plsc.store_compressed(crel.at[pl.ds(cnt, 16)], rel, mask=m)
            pc = plsc.all_reduce_population_count(m)
            return cnt + pc[0]

        def blk_body(blk, cnt):
            pltpu.sync_copy(src_hbm.at[pl.ds(estart + blk * SB, SB)], sv)
            pltpu.sync_copy(dst_hbm.at[pl.ds(estart + blk * SB, SB)], dv)
            cnt = lax.fori_loop(0, SB // 16, scan_vec, cnt)
            nfull = cnt // PB

            def grp_body(g, _):
                group(g * PB)
                return 0

            lax.fori_loop(0, nfull, grp_body, 0)
            rs = nfull * PB
            for j in range(PB // 16):
                v1 = csrc[pl.ds(rs + j * 16, 16)]
                v2 = crel[pl.ds(rs + j * 16, 16)]
                csrc[pl.ds(j * 16, 16)] = v1
                crel[pl.ds(j * 16, 16)] = v2
            return cnt - rs

        cnt = lax.fori_loop(0, BLKS, blk_body, 0)

        ones_m = jnp.ones((16,), jnp.bool_)
        zero_i = jnp.zeros((16,), jnp.int32)
        dum_r = jnp.full((16,), C, jnp.int32)
        plsc.store_compressed(csrc.at[pl.ds(cnt, 16)], zero_i, mask=ones_m)
        plsc.store_compressed(crel.at[pl.ds(cnt, 16)], dum_r, mask=ones_m)
        for j in range(PB // 16):

            @pl.when(j * 16 >= cnt)
            def _():
                csrc[pl.ds(j * 16, 16)] = zero_i
                crel[pl.ds(j * 16, 16)] = dum_r

        group(0)
        plsc.subcore_barrier()

        for t in range(4):
            pltpu.sync_copy(acc_m.at[pl.ds(r0 + t * PB, PB)],
                            acc_out.at[q, pl.ds(r0 + t * PB, PB)])
        pltpu.sync_copy(acc_m.at[pl.ds(r0 + 4 * PB, 16)],
                        acc_out.at[q, pl.ds(r0 + 4 * PB, 16)])
        plsc.subcore_barrier()


@functools.partial(
    pl.kernel,
    out_type=jax.ShapeDtypeStruct((NCHUNK, CP, FW), jnp.float32),
    mesh=plsc.VectorSubcoreMesh(core_axis_name="c", subcore_axis_name="s"),
    compiler_params=pltpu.CompilerParams(
        needs_layout_passes=False, use_tc_tiling_on_sc=False),
    scratch_types=(
        pltpu.VMEM_SHARED((CP, FW), jnp.float32),
        pltpu.VMEM((SB,), jnp.int32),
        pltpu.VMEM((SB,), jnp.int32),
        pltpu.VMEM((SB + 2 * PB,), jnp.int32),
        pltpu.VMEM((SB + 2 * PB,), jnp.int32),
        pltpu.VMEM((8, PB), jnp.int32),
        pltpu.VMEM((PB, HID), jnp.float32),
        pltpu.VMEM((PB, 16), jnp.float32),
        pltpu.VMEM((PB, 16), jnp.float32),
        pltpu.VMEM((PB, FW), jnp.float32),
        pltpu.SemaphoreType.DMA,
        pltpu.SemaphoreType.DMA,
        pltpu.SemaphoreType.DMA,
    ),
)
def _sc_conv(src_hbm, dst_hbm, h_hbm, as_hbm, ad_hbm, acc_out, *rest):
    _sc_conv_kernel(src_hbm, dst_hbm, h_hbm, as_hbm, ad_hbm,
                    acc_out, *rest)


def _proj_body(x_ref, w_ref, b_ref, a1_ref, a2_ref, h_ref, o1_ref, o2_ref):
    h = jnp.dot(x_ref[...], w_ref[...], preferred_element_type=jnp.float32)
    h = h + b_ref[...]
    h_ref[...] = h
    o1_ref[...] = jnp.dot(h, a1_ref[...], preferred_element_type=jnp.float32)
    o2_ref[...] = jnp.dot(h, a2_ref[...], preferred_element_type=jnp.float32)


def _proj(x, w, b, a1, a2):
    rb = 1000
    grid = (x.shape[0] // rb,)
    return pl.pallas_call(
        _proj_body,
        grid=grid,
        in_specs=[
            pl.BlockSpec((rb, HID), lambda i: (i, 0)),
            pl.BlockSpec((HID, HID), lambda i: (0, 0)),
            pl.BlockSpec((1, HID), lambda i: (0, 0)),
            pl.BlockSpec((HID, 16), lambda i: (0, 0)),
            pl.BlockSpec((HID, 16), lambda i: (0, 0)),
        ],
        out_specs=[
            pl.BlockSpec((rb, HID), lambda i: (i, 0)),
            pl.BlockSpec((rb, 16), lambda i: (i, 0)),
            pl.BlockSpec((rb, 16), lambda i: (i, 0)),
        ],
        out_shape=[
            jax.ShapeDtypeStruct((x.shape[0], HID), jnp.float32),
            jax.ShapeDtypeStruct((x.shape[0], 16), jnp.float32),
            jax.ShapeDtypeStruct((x.shape[0], 16), jnp.float32),
        ],
    )(x, w, b, a1, a2)


def _fin_body(a_ref, r_ref, o_ref):
    blk = a_ref[0]
    den16 = jnp.dot(blk[:, HID:HID + 16], r_ref[...],
                    preferred_element_type=jnp.float32)
    o_ref[0] = jnp.maximum(blk[:, :HID] / (den16 + 1e-16), 0.0)


def _finalize(acc3, rmat):
    rb = 704
    grid = (NCHUNK, CP // rb)
    out3 = pl.pallas_call(
        _fin_body,
        grid=grid,
        in_specs=[
            pl.BlockSpec((1, rb, FW), lambda i, j: (i, j, 0)),
            pl.BlockSpec((16, HID), lambda i, j: (0, 0)),
        ],
        out_specs=pl.BlockSpec((1, rb, HID), lambda i, j: (i, j, 0)),
        out_shape=jax.ShapeDtypeStruct((NCHUNK, CP, HID), jnp.float32),
    )(acc3, rmat)
    return out3[:, :C, :].reshape(NCHUNK * C, HID)[:N_NODE]


def _att_mat(att):
    blk = jnp.eye(HEADS, dtype=att.dtype)[:, None, :] * att[:, :, None]
    return jnp.pad(blk.reshape(HID, HEADS), ((0, 0), (0, 16 - HEADS)))


def _pad_edges(ei):
    src = jnp.concatenate(
        [ei[0], jnp.zeros((E_PAD - E_EDGE,), jnp.int32)])
    dst = jnp.concatenate(
        [ei[1], jnp.full((E_PAD - E_EDGE,), jnp.int32(2 ** 30))])
    return src, dst


def kernel(x_author, x_paper, edge_index_writes, edge_index_rev,
           W_proj_author, b_proj_author, W_proj_paper, b_proj_paper,
           att_src_writes, att_dst_writes, att_src_rev, att_dst_rev,
           q_sem, W_k_sem, b_k_sem):
    h_a, aa_w, aa_r = _proj(x_author, W_proj_author, b_proj_author.reshape(1, HID),
                            _att_mat(att_src_writes), _att_mat(att_dst_rev))
    h_p, ap_w, ap_r = _proj(x_paper, W_proj_paper, b_proj_paper.reshape(1, HID),
                            _att_mat(att_dst_writes), _att_mat(att_src_rev))

    src_w, dst_w = _pad_edges(edge_index_writes)
    src_r, dst_r = _pad_edges(edge_index_rev)

    acc_w = _sc_conv(src_w, dst_w, h_a, aa_w, ap_w)
    acc_r = _sc_conv(src_r, dst_r, h_p, ap_r, aa_r)

    rmat = jnp.pad(jnp.repeat(jnp.eye(HEADS, dtype=jnp.float32), 16, axis=1),
                   ((0, 16 - HEADS), (0, 0)))
    out_paper = _finalize(acc_w, rmat)
    out_author = _finalize(acc_r, rmat)
    return (out_author, out_paper)
